# baseline (speedup 1.0000x reference)
import jax
import jax.numpy as jnp
from jax.experimental import pallas as pl

N = 10000
H = 8
C = 16
G = 64


def _gat(x, edge_index, W, att_src, att_dst, bias):
    src = edge_index[0]
    dst = edge_index[1]
    h = (x @ W).reshape(N, H, C)
    a_src = (h * att_src[None, :, :]).sum(-1)
    a_dst = (h * att_dst[None, :, :]).sum(-1)
    e = a_src[src] + a_dst[dst]
    e = jnp.where(e > 0, e, 0.2 * e)
    m = jax.ops.segment_max(e, dst, num_segments=N)
    m = jnp.where(jnp.isfinite(m), m, 0.0)
    ex = jnp.exp(e - m[dst])
    denom = jax.ops.segment_sum(ex, dst, num_segments=N)
    alpha = ex / (denom[dst] + 1e-16)
    msg = h[src] * alpha[:, :, None]
    out = jax.ops.segment_sum(msg, dst, num_segments=N)
    return out.reshape(N, H * C) + bias


def _identity_pallas(x):
    def body(x_ref, o_ref):
        o_ref[...] = x_ref[...]

    return pl.pallas_call(body, out_shape=jax.ShapeDtypeStruct(x.shape, x.dtype))(x)


def kernel(x, edge_index, batch, W1, as1, ad1, b1, W2, as2, ad2, b2, W3, as3, ad3, b3, lw1, lb1, lw2, lb2):
    h = jax.nn.relu(_gat(x, edge_index, W1, as1, ad1, b1))
    h = jax.nn.relu(_gat(h, edge_index, W2, as2, ad2, b2))
    h = jax.nn.relu(_gat(h, edge_index, W3, as3, ad3, b3))
    sums = jax.ops.segment_sum(h, batch, num_segments=G)
    cnt = jax.ops.segment_sum(jnp.ones((N,), h.dtype), batch, num_segments=G)
    g = sums / jnp.clip(cnt, 1.0)[:, None]
    g = jax.nn.relu(g @ lw1 + lb1)
    return _identity_pallas(g @ lw2 + lb2)


# speedup vs baseline: 17.3502x; 17.3502x over previous
import dataclasses
import functools

import jax
import jax.numpy as jnp
from jax import lax
from jax.experimental import pallas as pl
from jax.experimental.pallas import tpu as pltpu
from jax.experimental.pallas import tpu_sc as plsc

N = 10000
E = 320000
D = 128
H = 8
C = 16
G = 64
OUT = 40

NC = 2
NS = 16
NW = NC * NS
NP = 10112
NPM = 10240
NR = NP // 8
NRM = NPM // 8
MW = NPM * H
PT = MW // NW
NT = PT // H
EW = 10240
EPAD = NW * EW
ROWS = EPAD // 128
PAD_DST = N + 7
CROWS = EW // 128


def _mesh():
    return plsc.VectorSubcoreMesh(core_axis_name="c", subcore_axis_name="s")


def _cp():
    cp = pltpu.CompilerParams()
    fields = pltpu.CompilerParams.__dataclass_fields__
    if "needs_layout_passes" in fields:
        cp = dataclasses.replace(cp, needs_layout_passes=False)
    if "use_tc_tiling_on_sc" in fields:
        cp = dataclasses.replace(cp, use_tc_tiling_on_sc=False)
    return cp


def _wid():
    return lax.axis_index("s") * NC + lax.axis_index("c")


def _sc_max(srcp, dstp, A, B):
    @functools.partial(
        pl.kernel,
        out_type=[
            jax.ShapeDtypeStruct((NW, MW), jnp.float32),
            jax.ShapeDtypeStruct((ROWS, 128, 16), jnp.float32),
        ],
        mesh=_mesh(),
        compiler_params=_cp(),
        scratch_types=[
            pltpu.VMEM((MW,), jnp.float32),
            pltpu.VMEM((128,), jnp.int32),
            pltpu.VMEM((128,), jnp.int32),
            pltpu.VMEM((128,), jnp.int32),
            pltpu.VMEM((128,), jnp.int32),
            pltpu.VMEM((128, 128), jnp.float32),
            pltpu.VMEM((128, 128), jnp.float32),
            pltpu.VMEM((128, 16), jnp.float32),
            pltpu.SemaphoreType.DMA,
        ],
    )
    def k(src_hbm, dst_hbm, a_hbm, b_hbm, o_hbm, e_hbm,
          mpart, srcb, dstb, srcb3, dstb3, arow, brow, ebuf, sem):
        w = _wid()
        iota = lax.iota(jnp.int32, 16)
        iot8 = jnp.bitwise_and(iota, 7)
        mask8 = iota < 8
        neginf = jnp.full((16,), -jnp.inf, jnp.float32)

        @pl.loop(0, MW, step=16)
        def _(i):
            mpart[pl.ds(i, 16)] = neginf

        row0 = w * CROWS

        @pl.loop(0, CROWS)
        def _(ch):
            rb = row0 + ch
            cps = [
                pltpu.async_copy(src_hbm.at[rb], srcb, sem),
                pltpu.async_copy(dst_hbm.at[rb], dstb, sem),
            ]
            for cp in cps:
                cp.wait()

            @pl.loop(0, 128, step=16)
            def _(i0):
                srcb3[pl.ds(i0, 16)] = srcb[pl.ds(i0, 16)] >> 3
                dstb3[pl.ds(i0, 16)] = dstb[pl.ds(i0, 16)] >> 3

            gs = [
                pltpu.async_copy(a_hbm.at[srcb3], arow, sem),
                pltpu.async_copy(b_hbm.at[dstb3], brow, sem),
            ]
            for cp in gs:
                cp.wait()

            @pl.loop(0, 128, step=16)
            def _(i0):
                sv = srcb[pl.ds(i0, 16)]
                dv = dstb[pl.ds(i0, 16)]
                for j in range(16):
                    i = i0 + j
                    s = sv[j]
                    d = dv[j]
                    av = arow[i, pl.ds((s & 7) * 16, 16)]
                    bv = brow[i, pl.ds((d & 7) * 16, 16)]
                    e = av + bv
                    e = jnp.maximum(e, 0.2 * e)
                    ebuf[i] = e
                    idx = jnp.full((16,), d * H, jnp.int32) + iot8
                    cur = plsc.load_gather(mpart, [idx], mask=mask8)
                    plsc.store_scatter(mpart, [idx], jnp.maximum(cur, e),
                                       mask=mask8)

            pltpu.sync_copy(ebuf, e_hbm.at[rb])

        pltpu.sync_copy(mpart, o_hbm.at[w])

    return k(srcp, dstp, A, B)


def _sc_merge(parts, op):
    @functools.partial(
        pl.kernel,
        out_type=jax.ShapeDtypeStruct((NPM, 16), jnp.float32),
        mesh=_mesh(),
        compiler_params=_cp(),
        scratch_types=[
            pltpu.VMEM((NW, PT), jnp.float32),
            pltpu.VMEM((NT, 16), jnp.float32),
            pltpu.SemaphoreType.DMA,
        ],
    )
    def k(p_hbm, o_hbm, slab, dup, sem):
        w = _wid()
        pltpu.sync_copy(p_hbm.at[:, pl.ds(w * PT, PT)], slab)
        iota = lax.iota(jnp.int32, 16)
        iot8 = jnp.bitwise_and(iota, 7)
        rowsel = jnp.where(iota >= 8, 1, 0)
        colb = iot8 + 8

        @pl.loop(0, PT, step=16)
        def _(j):
            acc = slab[0, pl.ds(j, 16)]
            for p in range(1, NW):
                v = slab[p, pl.ds(j, 16)]
                acc = jnp.maximum(acc, v) if op == "max" else acc + v
            if op == "inv":
                acc = 1.0 / (acc + 1e-16)
            rows = jnp.full((16,), j // 8, jnp.int32) + rowsel
            plsc.store_scatter(dup, [rows, iot8], acc)
            plsc.store_scatter(dup, [rows, colb], acc)

        pltpu.sync_copy(dup, o_hbm.at[pl.ds(w * NT, NT)])

    return k(parts)


def _sc_den(dstp, eh, M8):
    @functools.partial(
        pl.kernel,
        out_type=[
            jax.ShapeDtypeStruct((NW, MW), jnp.float32),
            jax.ShapeDtypeStruct((ROWS, 128, 16), jnp.float32),
        ],
        mesh=_mesh(),
        compiler_params=_cp(),
        scratch_types=[
            pltpu.VMEM((MW,), jnp.float32),
            pltpu.VMEM((128,), jnp.int32),
            pltpu.VMEM((128,), jnp.int32),
            pltpu.VMEM((128, 128), jnp.float32),
            pltpu.VMEM((128, 16), jnp.float32),
            pltpu.VMEM((128, 16), jnp.float32),
            pltpu.SemaphoreType.DMA,
        ],
    )
    def k(dst_hbm, e_hbm, m_hbm, o_hbm, ex_hbm,
          dpart, dstb, dstb3, mrow, ebuf, exb, sem):
        w = _wid()
        iota = lax.iota(jnp.int32, 16)
        iot8 = jnp.bitwise_and(iota, 7)
        mask8 = iota < 8
        zero = jnp.zeros((16,), jnp.float32)

        @pl.loop(0, MW, step=16)
        def _(i):
            dpart[pl.ds(i, 16)] = zero

        row0 = w * CROWS

        @pl.loop(0, CROWS)
        def _(ch):
            rb = row0 + ch
            cps = [
                pltpu.async_copy(dst_hbm.at[rb], dstb, sem),
                pltpu.async_copy(e_hbm.at[rb], ebuf, sem),
            ]
            for cp in cps:
                cp.wait()

            @pl.loop(0, 128, step=16)
            def _(i0):
                dstb3[pl.ds(i0, 16)] = dstb[pl.ds(i0, 16)] >> 3

            pltpu.async_copy(m_hbm.at[dstb3], mrow, sem).wait()

            @pl.loop(0, 128, step=16)
            def _(i0):
                dv = dstb[pl.ds(i0, 16)]
                for j in range(16):
                    i = i0 + j
                    d = dv[j]
                    mv = mrow[i, pl.ds((d & 7) * 16, 16)]
                    ex = jnp.exp(ebuf[i] - mv)
                    exb[i] = ex
                    idx = jnp.full((16,), d * H, jnp.int32) + iot8
                    cur = plsc.load_gather(dpart, [idx], mask=mask8)
                    plsc.store_scatter(dpart, [idx], cur + ex, mask=mask8)

            pltpu.sync_copy(exb, ex_hbm.at[rb])

        pltpu.sync_copy(dpart, o_hbm.at[w])

    return k(dstp, eh, M8)


def _sc_msg(srcp, dstp, ex, IV8, hmat, zeros):
    @functools.partial(
        pl.kernel,
        out_type=jax.ShapeDtypeStruct((NC, NP, 128), jnp.float32),
        mesh=_mesh(),
        compiler_params=_cp(),
        scratch_types=[
            pltpu.VMEM_SHARED((NP, 128), jnp.float32),
            pltpu.VMEM((128, 128), jnp.float32),
            pltpu.VMEM((128, 128), jnp.float32),
            pltpu.VMEM((128, 16), jnp.float32),
            pltpu.VMEM((128,), jnp.int32),
            pltpu.VMEM((128,), jnp.int32),
            pltpu.VMEM((128,), jnp.int32),
            pltpu.SemaphoreType.DMA,
        ],
    )
    def k(src_hbm, dst_hbm, ex_hbm, iv_hbm, h_hbm, z_hbm, o_hbm,
          out_sh, hbuf, ivrow, exb, srcb, dstb, dstb3, sem):
        sid = lax.axis_index("s")
        cid = lax.axis_index("c")
        w = sid * NC + cid

        r0 = sid * (NP // NS)
        pltpu.sync_copy(z_hbm.at[pl.ds(r0, NP // NS)],
                        out_sh.at[pl.ds(r0, NP // NS)])
        plsc.subcore_barrier()

        row0 = w * CROWS

        @pl.loop(0, CROWS)
        def _(ch):
            rb = row0 + ch
            cps = [
                pltpu.async_copy(src_hbm.at[rb], srcb, sem),
                pltpu.async_copy(dst_hbm.at[rb], dstb, sem),
                pltpu.async_copy(ex_hbm.at[rb], exb, sem),
            ]
            for cp in cps:
                cp.wait()

            @pl.loop(0, 128, step=16)
            def _(i0):
                dstb3[pl.ds(i0, 16)] = dstb[pl.ds(i0, 16)] >> 3

            gs = [
                pltpu.async_copy(h_hbm.at[srcb], hbuf, sem),
                pltpu.async_copy(iv_hbm.at[dstb3], ivrow, sem),
            ]
            for cp in gs:
                cp.wait()

            @pl.loop(0, 128, step=16)
            def _(i0):
                dv = dstb[pl.ds(i0, 16)]
                for j in range(16):
                    i = i0 + j
                    d = dv[j]
                    iv = ivrow[i, pl.ds((d & 7) * 16, 16)]
                    alpha = exb[i] * iv
                    for hh in range(H):
                        av = jnp.full((16,), alpha[hh], jnp.float32)
                        sl = pl.ds(hh * 16, 16)
                        hbuf[i, sl] = hbuf[i, sl] * av

            pltpu.sync_copy(hbuf, out_sh.at[dstb], add=True)

        plsc.subcore_barrier()
        pltpu.sync_copy(out_sh.at[pl.ds(r0, NP // NS)],
                        o_hbm.at[cid, pl.ds(r0, NP // NS)])

    return k(srcp, dstp, ex, IV8, hmat, zeros)


def _tc_front(xp, W, Ss, Sd):
    def body(x_ref, w_ref, ss_ref, sd_ref, h_ref, a_ref, b_ref):
        x = x_ref[...]
        h = jnp.dot(x, w_ref[...], preferred_element_type=jnp.float32)
        h_ref[...] = h
        asr = jnp.dot(h, ss_ref[...], preferred_element_type=jnp.float32)
        a_ref[...] = jnp.concatenate([asr, asr], axis=1)
        ads = jnp.dot(h, sd_ref[...], preferred_element_type=jnp.float32)
        b_ref[...] = jnp.concatenate([ads, ads], axis=1)

    return pl.pallas_call(
        body,
        out_shape=[
            jax.ShapeDtypeStruct((NP, 128), jnp.float32),
            jax.ShapeDtypeStruct((NP, 16), jnp.float32),
            jax.ShapeDtypeStruct((NP, 16), jnp.float32),
        ],
    )(xp, W, Ss, Sd)


def _tc_mid(parts, bias, W, Ss, Sd):
    def body(p_ref, bias_ref, w_ref, ss_ref, sd_ref, h_ref, a_ref, b_ref):
        x = jax.nn.relu(p_ref[0] + p_ref[1] + bias_ref[...])
        h = jnp.dot(x, w_ref[...], preferred_element_type=jnp.float32)
        h_ref[...] = h
        asr = jnp.dot(h, ss_ref[...], preferred_element_type=jnp.float32)
        a_ref[...] = jnp.concatenate([asr, asr], axis=1)
        ads = jnp.dot(h, sd_ref[...], preferred_element_type=jnp.float32)
        b_ref[...] = jnp.concatenate([ads, ads], axis=1)

    return pl.pallas_call(
        body,
        out_shape=[
            jax.ShapeDtypeStruct((NP, 128), jnp.float32),
            jax.ShapeDtypeStruct((NP, 16), jnp.float32),
            jax.ShapeDtypeStruct((NP, 16), jnp.float32),
        ],
    )(parts, bias, W, Ss, Sd)


def _tc_final(parts, bias, batchp, lw1, lb1, lw2, lb2):
    def body(p_ref, bias_ref, batch_ref, lw1_ref, lb1_ref, lw2_ref, lb2_ref, o_ref):
        h = jax.nn.relu(p_ref[0] + p_ref[1] + bias_ref[...])
        bidx = batch_ref[...]
        seg = lax.broadcasted_iota(jnp.int32, (NP, G), 1)
        oh = (bidx == seg).astype(jnp.float32)
        dn = (((0,), (0,)), ((), ()))
        sums = lax.dot_general(oh, h, dn, preferred_element_type=jnp.float32)
        ones = jnp.ones((NP, 1), jnp.float32)
        cnt = lax.dot_general(oh, ones, dn, preferred_element_type=jnp.float32)
        gm = sums / jnp.maximum(cnt, 1.0)
        g1 = jax.nn.relu(jnp.dot(gm, lw1_ref[...],
                                 preferred_element_type=jnp.float32) + lb1_ref[...])
        o_ref[...] = jnp.dot(g1, lw2_ref[...],
                             preferred_element_type=jnp.float32) + lb2_ref[...]

    return pl.pallas_call(
        body,
        out_shape=jax.ShapeDtypeStruct((G, OUT), jnp.float32),
    )(parts, bias, batchp, lw1, lb1, lw2, lb2)


def _smat(a):
    k = jnp.arange(D)
    msk = (k[:, None] // C) == jnp.arange(H)[None, :]
    return msk.astype(jnp.float32) * a.reshape(D)[:, None]


def _layer(xp_or_parts, bias_prev, W, a_s, a_d, srcp, dstp):
    Ss = _smat(a_s)
    Sd = _smat(a_d)
    if bias_prev is None:
        h, A, B = _tc_front(xp_or_parts, W, Ss, Sd)
    else:
        h, A, B = _tc_mid(xp_or_parts, bias_prev, W, Ss, Sd)
    A8 = A.reshape(NR, 128)
    B8 = B.reshape(NR, 128)
    m_parts, eh = _sc_max(srcp, dstp, A8, B8)
    M8 = _sc_merge(m_parts, "max").reshape(NRM, 128)
    den_parts, ex = _sc_den(dstp, eh, M8)
    IV8 = _sc_merge(den_parts, "inv").reshape(NRM, 128)
    return _sc_msg(srcp, dstp, ex, IV8, h, jnp.zeros((NP, 128), jnp.float32))


def kernel(x, edge_index, batch, W1, as1, ad1, b1, W2, as2, ad2, b2, W3,
           as3, ad3, b3, lw1, lb1, lw2, lb2):
    xp = jnp.pad(x, ((0, NP - N), (0, 0)))
    srcp = jnp.pad(edge_index[0], (0, EPAD - E)).reshape(ROWS, 128)
    dstp = jnp.pad(edge_index[1], (0, EPAD - E),
                   constant_values=PAD_DST).reshape(ROWS, 128)
    batchp = jnp.pad(batch, (0, NP - N), constant_values=G).reshape(NP, 1)

    p1 = _layer(xp, None, W1, as1, ad1, srcp, dstp)
    p2 = _layer(p1, b1.reshape(1, D), W2, as2, ad2, srcp, dstp)
    p3 = _layer(p2, b2.reshape(1, D), W3, as3, ad3, srcp, dstp)
    return _tc_final(p3, b3.reshape(1, D), batchp, lw1, lb1.reshape(1, 5 * OUT),
                     lw2, lb2.reshape(1, OUT))
